# baseline (speedup 1.0000x reference)
import functools

import numpy as np

import jax
import jax.numpy as jnp
from jax.experimental import pallas as pl
from jax.experimental.pallas import tpu as pltpu

_BM = 512
_KB = 1024


def _s1t_body(x_ref, w1_ref, o_ref):
    o_ref[...] = jax.lax.dot_general(
        w1_ref[...], x_ref[...], (((0,), (1,)), ((), ())),
        preferred_element_type=jnp.float32)


def _pass_a_body(adj_ref, s1t_ref, b1_ref, w2_ref, b2_ref,
                 s2_ref, part_ref, *, n, nstrips):
    i = pl.program_id(0)
    bm = adj_ref.shape[0]

    t = jax.lax.dot_general(
        adj_ref[...], s1t_ref[...], (((1,), (1,)), ((), ())),
        preferred_element_type=jnp.float32) + b1_ref[...]
    t = jnp.where(t >= 0, t, 0.01 * t)
    s2_ref[pl.ds(i * bm, bm), :] = jnp.dot(
        t, w2_ref[...], preferred_element_type=jnp.float32)

    @pl.when(i == nstrips - 1)
    def _():
        s2_ref[n:, :] = jnp.zeros_like(s2_ref[n:, :])

    part_ref[pl.ds(i * bm, bm), :] = jnp.broadcast_to(b2_ref[...], (bm, 16))
    for c in range(9):
        @pl.when(c < i // 2)
        def _(c=c):
            part_ref[pl.ds(i * bm, bm), :] += jnp.dot(
                adj_ref[:, c * _KB:(c + 1) * _KB],
                s2_ref[c * _KB:(c + 1) * _KB, :],
                preferred_element_type=jnp.float32)


def _pass_b_body(rref, cref, adj_ref, s2_ref, part_ref, o_ref, acc_ref,
                 *, n, n_nonedge, total):
    t = pl.program_id(0)
    bm = adj_ref.shape[0]

    @pl.when(t == 0)
    def _():
        acc_ref[...] = part_ref[...]

    r = rref[t]
    c = cref[t]

    @pl.when(t < n_nonedge)
    def _():
        acc_ref[pl.ds(r * bm, bm), :] += jnp.dot(
            adj_ref[...], s2_ref[pl.ds(c * _KB, _KB), :],
            preferred_element_type=jnp.float32)

    @pl.when(t >= n_nonedge)
    def _():
        w = n - 9 * _KB
        acc_ref[pl.ds(r * bm, bm), :] += jnp.dot(
            adj_ref[:, :w], s2_ref[9 * _KB:9 * _KB + w, :],
            preferred_element_type=jnp.float32)

    @pl.when(t == total - 1)
    def _():
        o_ref[...] = acc_ref[:n, :]


def kernel(x, adj, W1, b1, W2, b2):
    n, nfeat = x.shape
    nhid = W1.shape[1]
    bm = _BM
    nstrips = (n + bm - 1) // bm
    ns = nstrips * bm
    nkb = 9

    s1t = pl.pallas_call(
        _s1t_body,
        out_shape=jax.ShapeDtypeStruct((nhid, n), jnp.float32),
    )(x, W1)

    b1r = b1.reshape(1, nhid)
    b2r = b2.reshape(1, nhid)

    s2, part = pl.pallas_call(
        functools.partial(_pass_a_body, n=n, nstrips=nstrips),
        grid=(nstrips,),
        in_specs=[
            pl.BlockSpec((bm, n), lambda i: (i, 0)),
            pl.BlockSpec((nhid, n), lambda i: (0, 0)),
            pl.BlockSpec((1, nhid), lambda i: (0, 0)),
            pl.BlockSpec((nhid, nhid), lambda i: (0, 0)),
            pl.BlockSpec((1, nhid), lambda i: (0, 0)),
        ],
        out_specs=[
            pl.BlockSpec((ns, nhid), lambda i: (0, 0)),
            pl.BlockSpec((ns, nhid), lambda i: (0, 0)),
        ],
        out_shape=[
            jax.ShapeDtypeStruct((ns, nhid), jnp.float32),
            jax.ShapeDtypeStruct((ns, nhid), jnp.float32),
        ],
    )(adj, s1t, b1r, W2, b2r)

    nonedge = [(r, c) for r in range(nstrips) for c in range(r // 2, nkb)]
    edge = [(r, nkb) for r in range(nstrips)]
    pairs = nonedge + edge
    rarr = jnp.asarray(np.array([p[0] for p in pairs], dtype=np.int32))
    carr = jnp.asarray(np.array([p[1] for p in pairs], dtype=np.int32))
    total = len(pairs)

    out = pl.pallas_call(
        functools.partial(_pass_b_body, n=n, n_nonedge=len(nonedge),
                          total=total),
        grid_spec=pltpu.PrefetchScalarGridSpec(
            num_scalar_prefetch=2,
            grid=(total,),
            in_specs=[
                pl.BlockSpec((bm, _KB), lambda t, rr, cc: (rr[t], cc[t])),
                pl.BlockSpec((ns, nhid), lambda t, rr, cc: (0, 0)),
                pl.BlockSpec((ns, nhid), lambda t, rr, cc: (0, 0)),
            ],
            out_specs=pl.BlockSpec((n, nhid), lambda t, rr, cc: (0, 0)),
            scratch_shapes=[pltpu.VMEM((ns, nhid), jnp.float32)],
        ),
        out_shape=jax.ShapeDtypeStruct((n, nhid), jnp.float32),
    )(rarr, carr, adj, s2, part)
    return out


# speedup vs baseline: 1.2476x; 1.2476x over previous
import functools

import numpy as np

import jax
import jax.numpy as jnp
from jax.experimental import pallas as pl
from jax.experimental.pallas import tpu as pltpu

_BM = 512
_CB = 2048


def _s1_body(x_ref, w1_ref, o_ref):
    o_ref[...] = jnp.dot(x_ref[...], w1_ref[...],
                         preferred_element_type=jnp.float32)


def _pass_a_body(adj_ref, s1_ref, b1_ref, w2_ref, b2_ref,
                 s2_ref, part_ref, *, n, nstrips):
    i = pl.program_id(0)
    bm = adj_ref.shape[0]
    ns = s2_ref.shape[0]
    nhid = s1_ref.shape[1]

    kcov = _CB * (i // 4)
    row = jax.lax.broadcasted_iota(jnp.int32, (ns, nhid), 0)
    s2m = jnp.where(row < kcov, s2_ref[...], 0.0)[:n, :]
    rhs = jnp.concatenate([s1_ref[...], s2m], axis=1)

    res = jnp.dot(adj_ref[...], rhs, preferred_element_type=jnp.float32)

    t = res[:, :nhid] + b1_ref[...]
    t = jnp.where(t >= 0, t, 0.01 * t)
    s2_ref[pl.ds(i * bm, bm), :] = jnp.dot(
        t, w2_ref[...], preferred_element_type=jnp.float32)
    part_ref[pl.ds(i * bm, bm), :] = res[:, nhid:] + b2_ref[...]

    @pl.when(i == nstrips - 1)
    def _():
        s2_ref[n:, :] = jnp.zeros_like(s2_ref[n:, :])


def _pass_b_body(rref, cref, adj_ref, s2_ref, part_ref, o_ref, acc_ref,
                 *, n, n_nonedge, total):
    t = pl.program_id(0)
    cb = adj_ref.shape[0]

    @pl.when(t == 0)
    def _():
        acc_ref[...] = part_ref[...]

    r = rref[t]
    c = cref[t]

    @pl.when(t < n_nonedge)
    def _():
        acc_ref[pl.ds(r * cb, cb), :] += jnp.dot(
            adj_ref[...], s2_ref[pl.ds(c * _CB, _CB), :],
            preferred_element_type=jnp.float32)

    @pl.when(t >= n_nonedge)
    def _():
        w = n - 4 * _CB
        acc_ref[pl.ds(r * cb, cb), :] += jnp.dot(
            adj_ref[:, :w], s2_ref[4 * _CB:4 * _CB + w, :],
            preferred_element_type=jnp.float32)

    @pl.when(t == total - 1)
    def _():
        o_ref[...] = acc_ref[:n, :]


def kernel(x, adj, W1, b1, W2, b2):
    n, nfeat = x.shape
    nhid = W1.shape[1]
    bm = _BM
    nstrips = (n + bm - 1) // bm
    ns = nstrips * bm
    nkb = n // _CB

    s1 = pl.pallas_call(
        _s1_body,
        out_shape=jax.ShapeDtypeStruct((n, nhid), jnp.float32),
    )(x, W1)

    b1r = b1.reshape(1, nhid)
    b2r = b2.reshape(1, nhid)

    s2, part = pl.pallas_call(
        functools.partial(_pass_a_body, n=n, nstrips=nstrips),
        grid=(nstrips,),
        in_specs=[
            pl.BlockSpec((bm, n), lambda i: (i, 0)),
            pl.BlockSpec((n, nhid), lambda i: (0, 0)),
            pl.BlockSpec((1, nhid), lambda i: (0, 0)),
            pl.BlockSpec((nhid, nhid), lambda i: (0, 0)),
            pl.BlockSpec((1, nhid), lambda i: (0, 0)),
        ],
        out_specs=[
            pl.BlockSpec((ns, nhid), lambda i: (0, 0)),
            pl.BlockSpec((ns, nhid), lambda i: (0, 0)),
        ],
        out_shape=[
            jax.ShapeDtypeStruct((ns, nhid), jnp.float32),
            jax.ShapeDtypeStruct((ns, nhid), jnp.float32),
        ],
    )(adj, s1, b1r, W2, b2r)

    nrch = (n + _CB - 1) // _CB
    nonedge = [(r, c) for c in range(nkb) for r in range(min(c + 1, nrch))]
    edge = [(r, nkb) for r in range(nrch)]
    pairs = nonedge + edge
    rarr = jnp.asarray(np.array([p[0] for p in pairs], dtype=np.int32))
    carr = jnp.asarray(np.array([p[1] for p in pairs], dtype=np.int32))
    total = len(pairs)

    out = pl.pallas_call(
        functools.partial(_pass_b_body, n=n, n_nonedge=len(nonedge),
                          total=total),
        grid_spec=pltpu.PrefetchScalarGridSpec(
            num_scalar_prefetch=2,
            grid=(total,),
            in_specs=[
                pl.BlockSpec((_CB, _CB), lambda t, rr, cc: (rr[t], cc[t])),
                pl.BlockSpec((ns, nhid), lambda t, rr, cc: (0, 0)),
                pl.BlockSpec((ns, nhid), lambda t, rr, cc: (0, 0)),
            ],
            out_specs=pl.BlockSpec((n, nhid), lambda t, rr, cc: (0, 0)),
            scratch_shapes=[pltpu.VMEM((ns, nhid), jnp.float32)],
        ),
        out_shape=jax.ShapeDtypeStruct((n, nhid), jnp.float32),
    )(rarr, carr, adj, s2, part)
    return out


# speedup vs baseline: 1.2719x; 1.0195x over previous
import functools

import numpy as np

import jax
import jax.numpy as jnp
from jax.experimental import pallas as pl
from jax.experimental.pallas import tpu as pltpu

_BM = 512
_CB = 2048


def _s1_body(x_ref, w1_ref, o_ref):
    o_ref[...] = jnp.dot(x_ref[...], w1_ref[...],
                         preferred_element_type=jnp.float32)


def _pass_a_body(adj_ref, x_ref, w1_ref, b1_ref, w2_ref, b2_ref,
                 s2_ref, part_ref, s1_ref, *, n, nstrips):
    i = pl.program_id(0)

    @pl.when(i == 0)
    def _():
        s1_ref[...] = jnp.dot(x_ref[...], w1_ref[...],
                              preferred_element_type=jnp.float32)
    bm = adj_ref.shape[0]
    ns = s2_ref.shape[0]
    nhid = s1_ref.shape[1]

    kcov = _CB * (i // 4)
    row = jax.lax.broadcasted_iota(jnp.int32, (ns, nhid), 0)
    s2m = jnp.where(row < kcov, s2_ref[...], 0.0)[:n, :]
    rhs = jnp.concatenate([s1_ref[...], s2m], axis=1)

    res = jnp.dot(adj_ref[...], rhs, preferred_element_type=jnp.float32)

    t = res[:, :nhid] + b1_ref[...]
    t = jnp.where(t >= 0, t, 0.01 * t)
    s2_ref[pl.ds(i * bm, bm), :] = jnp.dot(
        t, w2_ref[...], preferred_element_type=jnp.float32)
    part_ref[pl.ds(i * bm, bm), :] = res[:, nhid:] + b2_ref[...]

    @pl.when(i == nstrips - 1)
    def _():
        s2_ref[n:, :] = jnp.zeros_like(s2_ref[n:, :])


def _pass_b_body(rref, cref, adj_ref, s2_ref, part_ref, o_ref, acc_ref,
                 *, n, n_nonedge, total):
    t = pl.program_id(0)
    cb = adj_ref.shape[0]

    @pl.when(t == 0)
    def _():
        acc_ref[...] = part_ref[...]

    r = rref[t]
    c = cref[t]

    @pl.when(t < n_nonedge)
    def _():
        acc_ref[pl.ds(r * cb, cb), :] += jnp.dot(
            adj_ref[...], s2_ref[pl.ds(c * _CB, _CB), :],
            preferred_element_type=jnp.float32)

    @pl.when(t >= n_nonedge)
    def _():
        w = n - 4 * _CB
        acc_ref[pl.ds(r * cb, cb), :] += jnp.dot(
            adj_ref[:, :w], s2_ref[4 * _CB:4 * _CB + w, :],
            preferred_element_type=jnp.float32)

    @pl.when(t == total - 1)
    def _():
        o_ref[...] = acc_ref[:n, :]


def kernel(x, adj, W1, b1, W2, b2):
    n, nfeat = x.shape
    nhid = W1.shape[1]
    bm = _BM
    nstrips = (n + bm - 1) // bm
    ns = nstrips * bm
    nkb = n // _CB

    b1r = b1.reshape(1, nhid)
    b2r = b2.reshape(1, nhid)

    s2, part = pl.pallas_call(
        functools.partial(_pass_a_body, n=n, nstrips=nstrips),
        grid=(nstrips,),
        in_specs=[
            pl.BlockSpec((bm, n), lambda i: (i, 0)),
            pl.BlockSpec((n, nfeat), lambda i: (0, 0)),
            pl.BlockSpec((nfeat, nhid), lambda i: (0, 0)),
            pl.BlockSpec((1, nhid), lambda i: (0, 0)),
            pl.BlockSpec((nhid, nhid), lambda i: (0, 0)),
            pl.BlockSpec((1, nhid), lambda i: (0, 0)),
        ],
        out_specs=[
            pl.BlockSpec((ns, nhid), lambda i: (0, 0)),
            pl.BlockSpec((ns, nhid), lambda i: (0, 0)),
        ],
        out_shape=[
            jax.ShapeDtypeStruct((ns, nhid), jnp.float32),
            jax.ShapeDtypeStruct((ns, nhid), jnp.float32),
        ],
        scratch_shapes=[pltpu.VMEM((n, nhid), jnp.float32)],
        compiler_params=pltpu.CompilerParams(
            vmem_limit_bytes=63 * 1024 * 1024),
    )(adj, x, W1, b1r, W2, b2r)

    nrch = (n + _CB - 1) // _CB
    nonedge = [(r, c) for c in range(nkb) for r in range(min(c + 1, nrch))]
    edge = [(r, nkb) for r in range(nrch)]
    pairs = nonedge + edge
    rarr = jnp.asarray(np.array([p[0] for p in pairs], dtype=np.int32))
    carr = jnp.asarray(np.array([p[1] for p in pairs], dtype=np.int32))
    total = len(pairs)

    out = pl.pallas_call(
        functools.partial(_pass_b_body, n=n, n_nonedge=len(nonedge),
                          total=total),
        grid_spec=pltpu.PrefetchScalarGridSpec(
            num_scalar_prefetch=2,
            grid=(total,),
            in_specs=[
                pl.BlockSpec((_CB, _CB), lambda t, rr, cc: (rr[t], cc[t])),
                pl.BlockSpec((ns, nhid), lambda t, rr, cc: (0, 0)),
                pl.BlockSpec((ns, nhid), lambda t, rr, cc: (0, 0)),
            ],
            out_specs=pl.BlockSpec((n, nhid), lambda t, rr, cc: (0, 0)),
            scratch_shapes=[pltpu.VMEM((ns, nhid), jnp.float32)],
        ),
        out_shape=jax.ShapeDtypeStruct((n, nhid), jnp.float32),
    )(rarr, carr, adj, s2, part)
    return out


# speedup vs baseline: 1.2748x; 1.0023x over previous
import functools

import numpy as np

import jax
import jax.numpy as jnp
from jax.experimental import pallas as pl
from jax.experimental.pallas import tpu as pltpu

_BM = 512
_CB = 2048


def _pass_a_body(adj_ref, x_ref, w1_ref, b1_ref, w2_ref, b2_ref,
                 s2_ref, part_ref, s1_ref, *, n, nstrips):
    i = pl.program_id(0)

    @pl.when(i == 0)
    def _():
        s1_ref[...] = jnp.dot(x_ref[...], w1_ref[...],
                              preferred_element_type=jnp.float32)
    bm = adj_ref.shape[0]
    ns = s2_ref.shape[0]
    nhid = s1_ref.shape[1]

    kcov = _CB * (i // 4)
    row = jax.lax.broadcasted_iota(jnp.int32, (ns, nhid), 0)
    s2m = jnp.where(row < kcov, s2_ref[...], 0.0)[:n, :]
    rhs = jnp.concatenate([s1_ref[...], s2m], axis=1)

    res = jnp.dot(adj_ref[...], rhs, preferred_element_type=jnp.float32)

    t = res[:, :nhid] + b1_ref[...]
    t = jnp.where(t >= 0, t, 0.01 * t)
    s2_ref[pl.ds(i * bm, bm), :] = jnp.dot(
        t, w2_ref[...], preferred_element_type=jnp.float32)
    part_ref[pl.ds(i * bm, bm), :] = res[:, nhid:] + b2_ref[...]

    @pl.when(i == nstrips - 1)
    def _():
        s2_ref[n:, :] = jnp.zeros_like(s2_ref[n:, :])


def _pass_b_body(rref, cref, adj_ref, s2_ref, part_ref, o_ref, acc_ref,
                 *, n, n_nonedge, total):
    t = pl.program_id(0)
    cb = adj_ref.shape[0]

    @pl.when(t == 0)
    def _():
        acc_ref[...] = part_ref[...]

    r = rref[t]
    c = cref[t]

    @pl.when(t < n_nonedge)
    def _():
        acc_ref[pl.ds(r * cb, cb), :] += jnp.dot(
            adj_ref[...], s2_ref[pl.ds(c * _CB, _CB), :],
            preferred_element_type=jnp.float32)

    @pl.when(t >= n_nonedge)
    def _():
        w = n - 4 * _CB
        acc_ref[pl.ds(r * cb, cb), :] += jnp.dot(
            adj_ref[:, :w], s2_ref[4 * _CB:4 * _CB + w, :],
            preferred_element_type=jnp.float32)

    @pl.when(t == total - 1)
    def _():
        o_ref[...] = acc_ref[:n, :]


def kernel(x, adj, W1, b1, W2, b2):
    n, nfeat = x.shape
    nhid = W1.shape[1]
    bm = _BM
    nstrips = (n + bm - 1) // bm
    ns = nstrips * bm
    nkb = n // _CB

    b1r = b1.reshape(1, nhid)
    b2r = b2.reshape(1, nhid)

    s2, part = pl.pallas_call(
        functools.partial(_pass_a_body, n=n, nstrips=nstrips),
        grid=(nstrips,),
        in_specs=[
            pl.BlockSpec((bm, n), lambda i: (i, 0)),
            pl.BlockSpec((n, nfeat), lambda i: (0, 0)),
            pl.BlockSpec((nfeat, nhid), lambda i: (0, 0)),
            pl.BlockSpec((1, nhid), lambda i: (0, 0)),
            pl.BlockSpec((nhid, nhid), lambda i: (0, 0)),
            pl.BlockSpec((1, nhid), lambda i: (0, 0)),
        ],
        out_specs=[
            pl.BlockSpec((ns, nhid), lambda i: (0, 0)),
            pl.BlockSpec((ns, nhid), lambda i: (0, 0)),
        ],
        out_shape=[
            jax.ShapeDtypeStruct((ns, nhid), jnp.float32),
            jax.ShapeDtypeStruct((ns, nhid), jnp.float32),
        ],
        scratch_shapes=[pltpu.VMEM((n, nhid), jnp.float32)],
        compiler_params=pltpu.CompilerParams(
            vmem_limit_bytes=63 * 1024 * 1024),
    )(adj, x, W1, b1r, W2, b2r)

    nrch = (n + _CB - 1) // _CB
    nonedge = [(r, c) for c in range(nkb) for r in range(min(c + 1, nrch))]
    edge = [(r, nkb) for r in range(nrch)]
    pairs = nonedge + edge
    rarr = jnp.asarray(np.array([p[0] for p in pairs], dtype=np.int32))
    carr = jnp.asarray(np.array([p[1] for p in pairs], dtype=np.int32))
    total = len(pairs)

    out = pl.pallas_call(
        functools.partial(_pass_b_body, n=n, n_nonedge=len(nonedge),
                          total=total),
        grid_spec=pltpu.PrefetchScalarGridSpec(
            num_scalar_prefetch=2,
            grid=(total,),
            in_specs=[
                pl.BlockSpec((_CB, _CB), lambda t, rr, cc: (rr[t], cc[t])),
                pl.BlockSpec((ns, nhid), lambda t, rr, cc: (0, 0)),
                pl.BlockSpec((ns, nhid), lambda t, rr, cc: (0, 0)),
            ],
            out_specs=pl.BlockSpec((n, nhid), lambda t, rr, cc: (0, 0)),
            scratch_shapes=[pltpu.VMEM((ns, nhid), jnp.float32)],
        ),
        out_shape=jax.ShapeDtypeStruct((n, nhid), jnp.float32),
    )(rarr, carr, adj, s2, part)
    return out
